# Initial kernel scaffold; baseline (speedup 1.0000x reference)
import jax
import jax.numpy as jnp
from jax.experimental import pallas as pl


def kernel(seq, W_region, W_word):
    raise NotImplementedError("write your pallas kernel here")


# speedup vs baseline: 3.6007x; 3.6007x over previous
import functools

import jax
import jax.numpy as jnp
from jax import lax
from jax.experimental import pallas as pl
from jax.experimental.pallas import tpu as pltpu
from jax.experimental.pallas import tpu_sc as plsc

V = 100000
WIN = 5
B = 1024
L = 200
EMB = 32
NWIN = L - WIN + 1

NC, NS = 2, 16
NWORK = NC * NS
SEQ_PER_W = B // NWORK

SEQ_PAD = 224
CHUNK = 112
NCH_R = WIN * SEQ_PAD // CHUNK
NCH_W = SEQ_PAD // CHUNK
NIDX = WIN * SEQ_PAD


def _body(seq_hbm, wr_hbm, ww_hbm, out_hbm, seq_pad, idx_v, rows_v, word_v, out_v, sem):
    wid = lax.axis_index("s") * NC + lax.axis_index("c")

    def per_seq(s, carry):
        b = wid * SEQ_PER_W + s
        zeros = jnp.zeros((16,), jnp.int32)
        seq_pad[pl.ds(L - 8, 16)] = zeros
        seq_pad[pl.ds(L + 8, 16)] = zeros
        pltpu.sync_copy(seq_hbm.at[pl.ds(b * L, L)], seq_pad.at[pl.ds(0, L)])

        for i in range(WIN):
            for k in range(SEQ_PAD // 16):
                idx_v[pl.ds(i * SEQ_PAD + k * 16, 16)] = (
                    seq_pad[pl.ds(k * 16, 16)] + (i * V)
                )

        copies = []
        for c in range(NCH_R):
            copies.append(pltpu.async_copy(
                wr_hbm.at[idx_v.at[pl.ds(c * CHUNK, CHUNK)]],
                rows_v.at[pl.ds(c * CHUNK, CHUNK)], sem))
        for c in range(NCH_W):
            copies.append(pltpu.async_copy(
                ww_hbm.at[seq_pad.at[pl.ds(c * CHUNK, CHUNK)]],
                word_v.at[pl.ds(c * CHUNK, CHUNK)], sem))
        for cp in copies:
            cp.wait()

        def win(p, c2):
            w0 = word_v[p + WIN // 2, pl.ds(0, 16)]
            w1 = word_v[p + WIN // 2, pl.ds(16, 16)]
            a0 = rows_v[p, pl.ds(0, 16)] * w0
            a1 = rows_v[p, pl.ds(16, 16)] * w1
            for i in range(1, WIN):
                r = p + i * SEQ_PAD + i
                a0 = jnp.maximum(a0, rows_v[r, pl.ds(0, 16)] * w0)
                a1 = jnp.maximum(a1, rows_v[r, pl.ds(16, 16)] * w1)
            out_v[pl.ds(p * EMB, 16)] = a0
            out_v[pl.ds(p * EMB + 16, 16)] = a1
            return c2

        lax.fori_loop(0, NWIN, win, 0)
        pltpu.sync_copy(out_v, out_hbm.at[pl.ds(b * (NWIN * EMB), NWIN * EMB)])
        return carry

    lax.fori_loop(0, SEQ_PER_W, per_seq, 0)


@jax.jit
def _run(seq, W_region, W_word):
    f = pl.kernel(
        _body,
        out_type=jax.ShapeDtypeStruct((B * NWIN * EMB,), jnp.float32),
        mesh=plsc.VectorSubcoreMesh(
            core_axis_name="c", subcore_axis_name="s",
            num_cores=NC, num_subcores=NS),
        scratch_types=[
            pltpu.VMEM((SEQ_PAD,), jnp.int32),
            pltpu.VMEM((NIDX,), jnp.int32),
            pltpu.VMEM((NIDX, EMB), jnp.float32),
            pltpu.VMEM((SEQ_PAD, EMB), jnp.float32),
            pltpu.VMEM((NWIN * EMB,), jnp.float32),
            pltpu.SemaphoreType.DMA,
        ],
        compiler_params=pltpu.CompilerParams(use_tc_tiling_on_sc=False),
    )
    out = f(seq.reshape(B * L), W_region, W_word)
    return out.reshape(B, NWIN, EMB)


def kernel(seq, W_region, W_word):
    return _run(seq.astype(jnp.int32), W_region, W_word)
